# baseline (speedup 1.0000x reference)
import math

import jax
import jax.numpy as jnp
from jax import lax
from jax.experimental import pallas as pl
from jax.experimental.pallas import tpu as pltpu
from jax.experimental.pallas import tpu_sc as plsc

_NA = 10000
_NB = 10000
_E = 160000
_D = 256
_H = 8
_DH = _D // _H
_INV_SQRT_DH = 1.0 / math.sqrt(_DH)
_ROW_BLK = 1000

_NC = 2
_NS = 16
_NW = _NC * _NS

_C1 = 80
_NCH1 = _E // _C1
_K1_BASE = _NCH1 // _NW
_K1_EXTRA = _NCH1 % _NW

_C2 = 80
_NCH2 = _E // _C2
_K2_BASE = _NCH2 // _NS
_K2_EXTRA = _NCH2 % _NS

_GW = 136
_RPT = _NA // _NS

_sc_mesh = plsc.VectorSubcoreMesh(
    core_axis_name="c", subcore_axis_name="s", num_cores=_NC, num_subcores=_NS)
_sc_params = pltpu.CompilerParams(
    needs_layout_passes=False, use_tc_tiling_on_sc=False)


def _qkv_body(fa_ref, fb_ref, wq_ref, wk_ref, wv_ref, q_ref, k_ref, v_ref):
    c = pl.program_id(1)

    @pl.when(c == 0)
    def _():
        q_ref[...] = fa_ref[...] @ wq_ref[...]
        k_ref[...] = fb_ref[...] @ wk_ref[...]

    v_ref[...] = fb_ref[...] @ wv_ref[...]


def _qkv(Fa, Fb, Wq, Wk, Wv):
    n_blk = _NA // _ROW_BLK
    row_spec = pl.BlockSpec((_ROW_BLK, _D), lambda i, c: (i, 0))
    w_spec = pl.BlockSpec((_D, _D), lambda i, c: (0, 0))
    wv_spec = pl.BlockSpec((_D, 128), lambda i, c: (0, c))
    v_spec = pl.BlockSpec((_ROW_BLK, 128), lambda i, c: (c * n_blk + i, 0))
    return pl.pallas_call(
        _qkv_body,
        grid=(n_blk, 2),
        in_specs=[row_spec, row_spec, w_spec, w_spec, wv_spec],
        out_specs=[row_spec, row_spec, v_spec],
        out_shape=[
            jax.ShapeDtypeStruct((_NA, _D), jnp.float32),
            jax.ShapeDtypeStruct((_NB, _D), jnp.float32),
            jax.ShapeDtypeStruct((2 * _NB, 128), jnp.float32),
        ],
    )(Fa, Fb, Wq, Wk, Wv)


def _k1_body(q_hbm, k_hbm, a_hbm, b_hbm, ex_hbm,
             av0, av1, bv0, bv1, qb0, qb1, kb0, kb1, exb0, exb1,
             isem0, isem1, gsem0, gsem1, ssem0, ssem1):
    isem = (isem0, isem1)
    gsem = (gsem0, gsem1)
    ssem = (ssem0, ssem1)
    cid = lax.axis_index("c")
    sid = lax.axis_index("s")
    wid = cid * _NS + sid
    nt = jnp.where(wid < _K1_EXTRA, _K1_BASE + 1, _K1_BASE)
    av = (av0, av1)
    bv = (bv0, bv1)
    qb = (qb0, qb1)
    kb = (kb0, kb1)
    exb = (exb0, exb1)

    lane = lax.iota(jnp.int32, 16)
    masks = [lane == i for i in range(16)]
    last_lane = jnp.full((16,), 15, jnp.int32)

    def coff(t):
        return (wid + _NW * t) * _C1

    def issue_idx(t, s):
        off = coff(t)
        pltpu.async_copy(a_hbm.at[pl.ds(off, _C1)], av[s], isem[s])
        pltpu.async_copy(b_hbm.at[pl.ds(off, _C1)], bv[s], isem[s])

    def wait_idx(t, s):
        off = coff(t)
        pltpu.make_async_copy(a_hbm.at[pl.ds(off, _C1)], av[s], isem[s]).wait()
        pltpu.make_async_copy(b_hbm.at[pl.ds(off, _C1)], bv[s], isem[s]).wait()

    def issue_gather(s):
        pltpu.async_copy(q_hbm.at[av[s]], qb[s], gsem[s])
        pltpu.async_copy(k_hbm.at[bv[s]], kb[s], gsem[s])

    def wait_gather(s):
        pltpu.make_async_copy(q_hbm.at[av[s]], qb[s], gsem[s]).wait()
        pltpu.make_async_copy(k_hbm.at[bv[s]], kb[s], gsem[s]).wait()

    def issue_store(t, s):
        off = coff(t)
        pltpu.async_copy(exb[s], ex_hbm.at[pl.ds(off * _H, _C1 * _H)], ssem[s])

    def wait_store(t, s):
        off = coff(t)
        pltpu.make_async_copy(
            exb[s], ex_hbm.at[pl.ds(off * _H, _C1 * _H)], ssem[s]).wait()

    issue_idx(0, 0)
    wait_idx(0, 0)
    issue_gather(0)
    issue_idx(1, 1)

    def iter_t(t, b):
        ob = 1 - b
        wait_gather(b)

        @pl.when(t >= 2)
        def _():
            wait_store(t - 2, b)

        @pl.when(t + 1 < nt)
        def _():
            wait_idx(t + 1, ob)
            issue_gather(ob)

        @pl.when(t + 2 < nt)
        def _():
            issue_idx(t + 2, b)

        qbb = qb[b]
        kbb = kb[b]
        exbb = exb[b]

        @plsc.parallel_loop(0, _C1 // 2, unroll=4)
        def pair_body(p):
            acc = jnp.zeros((16,), jnp.float32)
            for half in range(2):
                e = 2 * p + half
                for h in range(_H):
                    q0 = qbb[e, pl.ds(32 * h, 16)]
                    q1 = qbb[e, pl.ds(32 * h + 16, 16)]
                    k0 = kbb[e, pl.ds(32 * h, 16)]
                    k1 = kbb[e, pl.ds(32 * h + 16, 16)]
                    cs = jnp.cumsum(q0 * k0 + q1 * k1)
                    acc = jnp.where(masks[half * _H + h],
                                    jnp.take(cs, last_lane), acc)
            exbb[pl.ds(p * 16, 16)] = acc

        issue_store(t, b)

    def outer(u, carry):
        for b in range(2):
            t = 2 * u + b

            @pl.when(t < nt)
            def _():
                iter_t(t, b)
        return carry

    lax.fori_loop(0, (_K1_BASE + 2) // 2, outer, 0)
    wait_store(0, 0)
    wait_store(1, 1)


def _sc_logits(Q, K, a_idx, b_idx):
    f = pl.kernel(
        _k1_body,
        out_type=jax.ShapeDtypeStruct((_E * _H,), jnp.float32),
        mesh=_sc_mesh,
        compiler_params=_sc_params,
        scratch_types=(
            [pltpu.VMEM((_C1,), jnp.int32)] * 4
            + [pltpu.VMEM((_C1, _D), jnp.float32)] * 4
            + [pltpu.VMEM((_C1 * _H,), jnp.float32)] * 2
            + [pltpu.SemaphoreType.DMA] * 6
        ),
    )
    return f(Q, K, a_idx, b_idx)


def _k2_body(ex_hbm, a_hbm, b_hbm, vh_hbm, out_hbm,
             av0, av1, bv0, bv1, sav0, sav1, vb0, vb1, exb0, exb1,
             wvb0, wvb1, isem0, isem1, gsem0, gsem1, ssem0, ssem1, out_acc):
    isem = (isem0, isem1)
    gsem = (gsem0, gsem1)
    ssem = (ssem0, ssem1)
    cid = lax.axis_index("c")
    sid = lax.axis_index("s")
    nt = jnp.where(sid < _K2_EXTRA, _K2_BASE + 1, _K2_BASE)
    av = (av0, av1)
    bv = (bv0, bv1)
    sav = (sav0, sav1)
    vb = (vb0, vb1)
    exb = (exb0, exb1)
    wvb = (wvb0, wvb1)

    def zrow(r, carry):
        for j in range(8):
            wvb0[r, pl.ds(16 * j, 16)] = jnp.zeros((16,), jnp.float32)
        wvb0[r, pl.ds(_GW - 16, 16)] = jnp.zeros((16,), jnp.float32)
        return carry

    lax.fori_loop(0, _C2, zrow, 0)
    r0 = sid * _RPT
    off2 = 0
    for sz in (80,) * 7 + (65,):
        pltpu.sync_copy(wvb0.at[pl.ds(0, sz)], out_acc.at[pl.ds(r0 + off2, sz)])
        off2 += sz
    plsc.subcore_barrier()

    def coff(t):
        return (sid + _NS * t) * _C2

    def issue_idx(t, s):
        off = coff(t)
        pltpu.async_copy(a_hbm.at[pl.ds(off, _C2)], av[s], isem[s])
        pltpu.async_copy(b_hbm.at[pl.ds(off, _C2)], bv[s], isem[s])

    def wait_idx(t, s):
        off = coff(t)
        pltpu.make_async_copy(a_hbm.at[pl.ds(off, _C2)], av[s], isem[s]).wait()
        pltpu.make_async_copy(b_hbm.at[pl.ds(off, _C2)], bv[s], isem[s]).wait()

    def issue_gather(t, s):
        off = coff(t)

        def addb(i, c2):
            bv[s][pl.ds(i * 16, 16)] = bv[s][pl.ds(i * 16, 16)] + cid * _NB
            return c2

        lax.fori_loop(0, _C2 // 16, addb, 0)
        pltpu.async_copy(vh_hbm.at[bv[s]], vb[s], gsem[s])
        pltpu.async_copy(ex_hbm.at[pl.ds(off * _H, _C2 * _H)],
                         exb[s].at[pl.ds(0, _C2 * _H)], gsem[s])

    def wait_gather(t, s):
        off = coff(t)
        pltpu.make_async_copy(vh_hbm.at[bv[s]], vb[s], gsem[s]).wait()
        pltpu.make_async_copy(ex_hbm.at[pl.ds(off * _H, _C2 * _H)],
                              exb[s].at[pl.ds(0, _C2 * _H)], gsem[s]).wait()

    def issue_scatter(s):
        pltpu.async_copy(wvb[s], out_acc.at[sav[s]], ssem[s], add=True)

    def wait_scatter(s):
        pltpu.make_async_copy(wvb[s], out_acc.at[sav[s]], ssem[s]).wait()

    issue_idx(0, 0)
    wait_idx(0, 0)
    issue_gather(0, 0)
    issue_idx(1, 1)

    def iter_t(t, b):
        ob = 1 - b
        wait_gather(t, b)

        @pl.when(t >= 2)
        def _():
            wait_scatter(b)

        def cpav(i, c2):
            sav[b][pl.ds(i * 16, 16)] = av[b][pl.ds(i * 16, 16)]
            return c2

        lax.fori_loop(0, _C2 // 16, cpav, 0)

        @pl.when(t + 1 < nt)
        def _():
            wait_idx(t + 1, ob)
            issue_gather(t + 1, ob)

        @pl.when(t + 2 < nt)
        def _():
            issue_idx(t + 2, b)

        exbb = exb[b]
        vbb = vb[b]
        wvbb = wvb[b]

        @plsc.parallel_loop(0, _C2 * _H // 16, unroll=4)
        def exp_body(t2):
            v = exbb[pl.ds(t2 * 16, 16)]
            exbb[pl.ds(t2 * 16, 16)] = jnp.exp(v * _INV_SQRT_DH)

        bidx = [jnp.full((16,), i, jnp.int32) for i in range(4)]

        @plsc.parallel_loop(0, _C2, unroll=4)
        def edge_body(e):
            wvbb[e, pl.ds(0, 16)] = exbb[pl.ds(e * _H, 16)]
            exv = exbb[pl.ds(e * _H + 4 * cid, 16)]
            mv = [jnp.take(exv, bidx[i]) for i in range(4)]
            for j in range(8):
                wvbb[e, pl.ds(8 + 16 * j, 16)] = (
                    vbb[e, pl.ds(16 * j, 16)] * mv[j // 2])

        issue_scatter(b)

    def outer(u, carry):
        for b in range(2):
            t = 2 * u + b

            @pl.when(t < nt)
            def _():
                iter_t(t, b)
        return carry

    lax.fori_loop(0, (_K2_BASE + 2) // 2, outer, 0)
    wait_scatter(0)
    wait_scatter(1)
    plsc.subcore_barrier()

    off2 = 0
    for sz in (80,) * 7 + (65,):
        pltpu.sync_copy(out_acc.at[pl.ds(r0 + off2, sz)], wvb0.at[pl.ds(0, sz)])
        pltpu.sync_copy(wvb0.at[pl.ds(0, sz)],
                        out_hbm.at[cid, pl.ds(r0 + off2, sz)])
        off2 += sz


def _sc_scatter(ex, a_idx, b_idx, Vh):
    f = pl.kernel(
        _k2_body,
        out_type=jax.ShapeDtypeStruct((2, _NA, _GW), jnp.float32),
        mesh=_sc_mesh,
        compiler_params=_sc_params,
        scratch_types=(
            [pltpu.VMEM((_C2,), jnp.int32)] * 6
            + [pltpu.VMEM((_C2, 128), jnp.float32)] * 2
            + [pltpu.VMEM((_C2 * _H + 16,), jnp.float32)] * 2
            + [pltpu.VMEM((_C2, _GW), jnp.float32)] * 2
            + [pltpu.SemaphoreType.DMA] * 6
            + [pltpu.VMEM_SHARED((_NA, _GW), jnp.float32)]
        ),
    )
    return f(ex, a_idx, b_idx, Vh)


def _final_body(fa_ref, g0_ref, g1_ref, wp_ref, y_ref):
    g0 = g0_ref[0]
    g1 = g1_ref[0]
    s = g0[:, 0:8]
    s = jnp.where(s > 0.0, s, 1.0)
    o = jnp.concatenate([g0[:, 8:_GW], g1[:, 8:_GW]], axis=1)
    o = (o.reshape(_ROW_BLK, _H, _DH) / s[:, :, None]).reshape(_ROW_BLK, _D)
    y_ref[...] = fa_ref[...] + o @ wp_ref[...]


def _final(Fa, g, Wp):
    n_blk = _NA // _ROW_BLK
    row_spec = pl.BlockSpec((_ROW_BLK, _D), lambda i: (i, 0))
    g0_spec = pl.BlockSpec((1, _ROW_BLK, _GW), lambda i: (0, i, 0))
    g1_spec = pl.BlockSpec((1, _ROW_BLK, _GW), lambda i: (1, i, 0))
    w_spec = pl.BlockSpec((_D, _D), lambda i: (0, 0))
    return pl.pallas_call(
        _final_body,
        grid=(n_blk,),
        in_specs=[row_spec, g0_spec, g1_spec, w_spec],
        out_specs=row_spec,
        out_shape=jax.ShapeDtypeStruct((_NA, _D), jnp.float32),
    )(Fa, g, g, Wp)


def kernel(Fa, Fb, a_idx, b_idx, Wq, Wk, Wv, Wp):
    a_idx = a_idx.astype(jnp.int32)
    b_idx = b_idx.astype(jnp.int32)
    Q, K, Vh = _qkv(Fa, Fb, Wq, Wk, Wv)
    ex = _sc_logits(Q, K, a_idx, b_idx)
    g = _sc_scatter(ex, a_idx, b_idx, Vh)
    return _final(Fa, g, Wp)


# speedup vs baseline: 1.1397x; 1.1397x over previous
import math

import jax
import jax.numpy as jnp
from jax import lax
from jax.experimental import pallas as pl
from jax.experimental.pallas import tpu as pltpu
from jax.experimental.pallas import tpu_sc as plsc

_NA = 10000
_NB = 10000
_E = 160000
_D = 256
_H = 8
_DH = _D // _H
_INV_SQRT_DH = 1.0 / math.sqrt(_DH)
_ROW_BLK = 1000

_NC = 2
_NS = 16
_NW = _NC * _NS

_C1 = 80
_NCH1 = _E // _C1
_K1_BASE = _NCH1 // _NW
_K1_EXTRA = _NCH1 % _NW

_C2 = 80
_NCH2 = _E // _C2
_K2_BASE = _NCH2 // _NS
_K2_EXTRA = _NCH2 % _NS

_GW = 136
_RPT = _NA // _NS

_sc_mesh = plsc.VectorSubcoreMesh(
    core_axis_name="c", subcore_axis_name="s", num_cores=_NC, num_subcores=_NS)
_sc_params = pltpu.CompilerParams(
    needs_layout_passes=False, use_tc_tiling_on_sc=False)


def _qkv_body(fa_ref, fb_ref, wq_ref, wk_ref, wv_ref, q_ref, k_ref, v_ref):
    c = pl.program_id(1)

    @pl.when(c == 0)
    def _():
        q_ref[...] = fa_ref[...] @ wq_ref[...]
        k_ref[...] = fb_ref[...] @ wk_ref[...]

    v_ref[...] = fb_ref[...] @ wv_ref[...]


def _qkv(Fa, Fb, Wq, Wk, Wv):
    n_blk = _NA // _ROW_BLK
    row_spec = pl.BlockSpec((_ROW_BLK, _D), lambda i, c: (i, 0))
    w_spec = pl.BlockSpec((_D, _D), lambda i, c: (0, 0))
    wv_spec = pl.BlockSpec((_D, 128), lambda i, c: (0, c))
    v_spec = pl.BlockSpec((_ROW_BLK, 128), lambda i, c: (c * n_blk + i, 0))
    return pl.pallas_call(
        _qkv_body,
        grid=(n_blk, 2),
        in_specs=[row_spec, row_spec, w_spec, w_spec, wv_spec],
        out_specs=[row_spec, row_spec, v_spec],
        out_shape=[
            jax.ShapeDtypeStruct((_NA, _D), jnp.float32),
            jax.ShapeDtypeStruct((_NB, _D), jnp.float32),
            jax.ShapeDtypeStruct((2 * _NB, 128), jnp.float32),
        ],
    )(Fa, Fb, Wq, Wk, Wv)


def _k1_body(q_hbm, k_hbm, a_hbm, b_hbm, ex_hbm,
             av0, av1, bv0, bv1, qb0, qb1, kb0, kb1, exb0, exb1,
             isem0, isem1, gsem0, gsem1, ssem0, ssem1):
    isem = (isem0, isem1)
    gsem = (gsem0, gsem1)
    ssem = (ssem0, ssem1)
    cid = lax.axis_index("c")
    sid = lax.axis_index("s")
    wid = cid * _NS + sid
    nt = jnp.where(wid < _K1_EXTRA, _K1_BASE + 1, _K1_BASE)
    av = (av0, av1)
    bv = (bv0, bv1)
    qb = (qb0, qb1)
    kb = (kb0, kb1)
    exb = (exb0, exb1)

    lane = lax.iota(jnp.int32, 16)
    masks = [lane == i for i in range(16)]
    last_lane = jnp.full((16,), 15, jnp.int32)

    def coff(t):
        return (wid + _NW * t) * _C1

    def issue_idx(t, s):
        off = coff(t)
        pltpu.async_copy(a_hbm.at[pl.ds(off, _C1)], av[s], isem[s])
        pltpu.async_copy(b_hbm.at[pl.ds(off, _C1)], bv[s], isem[s])

    def wait_idx(t, s):
        off = coff(t)
        pltpu.make_async_copy(a_hbm.at[pl.ds(off, _C1)], av[s], isem[s]).wait()
        pltpu.make_async_copy(b_hbm.at[pl.ds(off, _C1)], bv[s], isem[s]).wait()

    def issue_gather(s):
        pltpu.async_copy(q_hbm.at[av[s]], qb[s], gsem[s])
        pltpu.async_copy(k_hbm.at[bv[s]], kb[s], gsem[s])

    def wait_gather(s):
        pltpu.make_async_copy(q_hbm.at[av[s]], qb[s], gsem[s]).wait()
        pltpu.make_async_copy(k_hbm.at[bv[s]], kb[s], gsem[s]).wait()

    def issue_store(t, s):
        off = coff(t)
        pltpu.async_copy(exb[s], ex_hbm.at[pl.ds(off * _H, _C1 * _H)], ssem[s])

    def wait_store(t, s):
        off = coff(t)
        pltpu.make_async_copy(
            exb[s], ex_hbm.at[pl.ds(off * _H, _C1 * _H)], ssem[s]).wait()

    issue_idx(0, 0)
    wait_idx(0, 0)
    issue_gather(0)
    issue_idx(1, 1)

    def iter_t(t, b):
        ob = 1 - b
        wait_gather(b)

        @pl.when(t >= 2)
        def _():
            wait_store(t - 2, b)

        @pl.when(t + 1 < nt)
        def _():
            wait_idx(t + 1, ob)
            issue_gather(ob)

        @pl.when(t + 2 < nt)
        def _():
            issue_idx(t + 2, b)

        qbb = qb[b]
        kbb = kb[b]
        exbb = exb[b]

        @plsc.parallel_loop(0, _C1 // 2, unroll=2)
        def pair_body(p):
            acc = jnp.zeros((16,), jnp.float32)
            for half in range(2):
                e = 2 * p + half
                for h in range(_H):
                    q0 = qbb[e, pl.ds(32 * h, 16)]
                    q1 = qbb[e, pl.ds(32 * h + 16, 16)]
                    k0 = kbb[e, pl.ds(32 * h, 16)]
                    k1 = kbb[e, pl.ds(32 * h + 16, 16)]
                    cs = jnp.cumsum(q0 * k0 + q1 * k1)
                    acc = jnp.where(masks[half * _H + h],
                                    jnp.take(cs, last_lane), acc)
            exbb[pl.ds(p * 16, 16)] = acc

        issue_store(t, b)

    def outer(u, carry):
        for b in range(2):
            t = 2 * u + b

            @pl.when(t < nt)
            def _():
                iter_t(t, b)
        return carry

    lax.fori_loop(0, (_K1_BASE + 2) // 2, outer, 0)
    wait_store(0, 0)
    wait_store(1, 1)


def _sc_logits(Q, K, a_idx, b_idx):
    f = pl.kernel(
        _k1_body,
        out_type=jax.ShapeDtypeStruct((_E * _H,), jnp.float32),
        mesh=_sc_mesh,
        compiler_params=_sc_params,
        scratch_types=(
            [pltpu.VMEM((_C1,), jnp.int32)] * 4
            + [pltpu.VMEM((_C1, _D), jnp.float32)] * 4
            + [pltpu.VMEM((_C1 * _H,), jnp.float32)] * 2
            + [pltpu.SemaphoreType.DMA] * 6
        ),
    )
    return f(Q, K, a_idx, b_idx)


def _k2_body(ex_hbm, a_hbm, b_hbm, vh_hbm, out_hbm,
             av0, av1, bv0, bv1, sav0, sav1, vb0, vb1, exb0, exb1,
             wvb0, wvb1, isem0, isem1, gsem0, gsem1, ssem0, ssem1, out_acc):
    isem = (isem0, isem1)
    gsem = (gsem0, gsem1)
    ssem = (ssem0, ssem1)
    cid = lax.axis_index("c")
    sid = lax.axis_index("s")
    nt = jnp.where(sid < _K2_EXTRA, _K2_BASE + 1, _K2_BASE)
    av = (av0, av1)
    bv = (bv0, bv1)
    sav = (sav0, sav1)
    vb = (vb0, vb1)
    exb = (exb0, exb1)
    wvb = (wvb0, wvb1)

    def zrow(r, carry):
        for j in range(8):
            wvb0[r, pl.ds(16 * j, 16)] = jnp.zeros((16,), jnp.float32)
        wvb0[r, pl.ds(_GW - 16, 16)] = jnp.zeros((16,), jnp.float32)
        return carry

    lax.fori_loop(0, _C2, zrow, 0)
    r0 = sid * _RPT
    off2 = 0
    for sz in (80,) * 7 + (65,):
        pltpu.sync_copy(wvb0.at[pl.ds(0, sz)], out_acc.at[pl.ds(r0 + off2, sz)])
        off2 += sz
    plsc.subcore_barrier()

    def coff(t):
        return (sid + _NS * t) * _C2

    def issue_idx(t, s):
        off = coff(t)
        pltpu.async_copy(a_hbm.at[pl.ds(off, _C2)], av[s], isem[s])
        pltpu.async_copy(b_hbm.at[pl.ds(off, _C2)], bv[s], isem[s])

    def wait_idx(t, s):
        off = coff(t)
        pltpu.make_async_copy(a_hbm.at[pl.ds(off, _C2)], av[s], isem[s]).wait()
        pltpu.make_async_copy(b_hbm.at[pl.ds(off, _C2)], bv[s], isem[s]).wait()

    def issue_gather(t, s):
        off = coff(t)

        def addb(i, c2):
            bv[s][pl.ds(i * 16, 16)] = bv[s][pl.ds(i * 16, 16)] + cid * _NB
            return c2

        lax.fori_loop(0, _C2 // 16, addb, 0)
        pltpu.async_copy(vh_hbm.at[bv[s]], vb[s], gsem[s])
        pltpu.async_copy(ex_hbm.at[pl.ds(off * _H, _C2 * _H)],
                         exb[s].at[pl.ds(0, _C2 * _H)], gsem[s])

    def wait_gather(t, s):
        off = coff(t)
        pltpu.make_async_copy(vh_hbm.at[bv[s]], vb[s], gsem[s]).wait()
        pltpu.make_async_copy(ex_hbm.at[pl.ds(off * _H, _C2 * _H)],
                              exb[s].at[pl.ds(0, _C2 * _H)], gsem[s]).wait()

    def issue_scatter(s):
        pltpu.async_copy(wvb[s], out_acc.at[sav[s]], ssem[s], add=True)

    def wait_scatter(s):
        pltpu.make_async_copy(wvb[s], out_acc.at[sav[s]], ssem[s]).wait()

    issue_idx(0, 0)
    wait_idx(0, 0)
    issue_gather(0, 0)
    issue_idx(1, 1)

    def iter_t(t, b):
        ob = 1 - b
        wait_gather(t, b)

        @pl.when(t >= 2)
        def _():
            wait_scatter(b)

        def cpav(i, c2):
            sav[b][pl.ds(i * 16, 16)] = av[b][pl.ds(i * 16, 16)]
            return c2

        lax.fori_loop(0, _C2 // 16, cpav, 0)

        @pl.when(t + 1 < nt)
        def _():
            wait_idx(t + 1, ob)
            issue_gather(t + 1, ob)

        @pl.when(t + 2 < nt)
        def _():
            issue_idx(t + 2, b)

        exbb = exb[b]
        vbb = vb[b]
        wvbb = wvb[b]

        @plsc.parallel_loop(0, _C2 * _H // 16, unroll=4)
        def exp_body(t2):
            v = exbb[pl.ds(t2 * 16, 16)]
            exbb[pl.ds(t2 * 16, 16)] = jnp.exp(v * _INV_SQRT_DH)

        bidx = [jnp.full((16,), i, jnp.int32) for i in range(4)]

        @plsc.parallel_loop(0, _C2, unroll=2)
        def edge_body(e):
            wvbb[e, pl.ds(0, 16)] = exbb[pl.ds(e * _H, 16)]
            exv = exbb[pl.ds(e * _H + 4 * cid, 16)]
            mv = [jnp.take(exv, bidx[i]) for i in range(4)]
            for j in range(8):
                wvbb[e, pl.ds(8 + 16 * j, 16)] = (
                    vbb[e, pl.ds(16 * j, 16)] * mv[j // 2])

        issue_scatter(b)

    def outer(u, carry):
        for b in range(2):
            t = 2 * u + b

            @pl.when(t < nt)
            def _():
                iter_t(t, b)
        return carry

    lax.fori_loop(0, (_K2_BASE + 2) // 2, outer, 0)
    wait_scatter(0)
    wait_scatter(1)
    plsc.subcore_barrier()

    off2 = 0
    for sz in (80,) * 7 + (65,):
        pltpu.sync_copy(out_acc.at[pl.ds(r0 + off2, sz)], wvb0.at[pl.ds(0, sz)])
        pltpu.sync_copy(wvb0.at[pl.ds(0, sz)],
                        out_hbm.at[cid, pl.ds(r0 + off2, sz)])
        off2 += sz


def _sc_scatter(ex, a_idx, b_idx, Vh):
    f = pl.kernel(
        _k2_body,
        out_type=jax.ShapeDtypeStruct((2, _NA, _GW), jnp.float32),
        mesh=_sc_mesh,
        compiler_params=_sc_params,
        scratch_types=(
            [pltpu.VMEM((_C2,), jnp.int32)] * 6
            + [pltpu.VMEM((_C2, 128), jnp.float32)] * 2
            + [pltpu.VMEM((_C2 * _H + 16,), jnp.float32)] * 2
            + [pltpu.VMEM((_C2, _GW), jnp.float32)] * 2
            + [pltpu.SemaphoreType.DMA] * 6
            + [pltpu.VMEM_SHARED((_NA, _GW), jnp.float32)]
        ),
    )
    return f(ex, a_idx, b_idx, Vh)


def _final_body(fa_ref, g0_ref, g1_ref, wp_ref, y_ref):
    g0 = g0_ref[0]
    g1 = g1_ref[0]
    s = g0[:, 0:8]
    s = jnp.where(s > 0.0, s, 1.0)
    o = jnp.concatenate([g0[:, 8:_GW], g1[:, 8:_GW]], axis=1)
    o = (o.reshape(_ROW_BLK, _H, _DH) / s[:, :, None]).reshape(_ROW_BLK, _D)
    y_ref[...] = fa_ref[...] + o @ wp_ref[...]


def _final(Fa, g, Wp):
    n_blk = _NA // _ROW_BLK
    row_spec = pl.BlockSpec((_ROW_BLK, _D), lambda i: (i, 0))
    g0_spec = pl.BlockSpec((1, _ROW_BLK, _GW), lambda i: (0, i, 0))
    g1_spec = pl.BlockSpec((1, _ROW_BLK, _GW), lambda i: (1, i, 0))
    w_spec = pl.BlockSpec((_D, _D), lambda i: (0, 0))
    return pl.pallas_call(
        _final_body,
        grid=(n_blk,),
        in_specs=[row_spec, g0_spec, g1_spec, w_spec],
        out_specs=row_spec,
        out_shape=jax.ShapeDtypeStruct((_NA, _D), jnp.float32),
    )(Fa, g, g, Wp)


def kernel(Fa, Fb, a_idx, b_idx, Wq, Wk, Wv, Wp):
    a_idx = a_idx.astype(jnp.int32)
    b_idx = b_idx.astype(jnp.int32)
    Q, K, Vh = _qkv(Fa, Fb, Wq, Wk, Wv)
    ex = _sc_logits(Q, K, a_idx, b_idx)
    g = _sc_scatter(ex, a_idx, b_idx, Vh)
    return _final(Fa, g, Wp)


# speedup vs baseline: 1.2049x; 1.0573x over previous
import math

import jax
import jax.numpy as jnp
from jax import lax
from jax.experimental import pallas as pl
from jax.experimental.pallas import tpu as pltpu
from jax.experimental.pallas import tpu_sc as plsc

_NA = 10000
_NB = 10000
_E = 160000
_D = 256
_H = 8
_DH = _D // _H
_INV_SQRT_DH = 1.0 / math.sqrt(_DH)
_ROW_BLK = 1000

_NC = 2
_NS = 16
_NW = _NC * _NS

_C1 = 80
_NCH1 = _E // _C1
_K1_BASE = _NCH1 // _NW
_K1_EXTRA = _NCH1 % _NW

_C2 = 80
_NCH2 = _E // _C2
_K2_BASE = _NCH2 // _NS
_K2_EXTRA = _NCH2 % _NS

_GW = 136
_RPT = _NA // _NS

_sc_mesh = plsc.VectorSubcoreMesh(
    core_axis_name="c", subcore_axis_name="s", num_cores=_NC, num_subcores=_NS)
_sc_params = pltpu.CompilerParams(
    needs_layout_passes=False, use_tc_tiling_on_sc=False)
_sc_params_tiled = pltpu.CompilerParams(
    needs_layout_passes=False, use_tc_tiling_on_sc=True)


def _qkv_body(fa_ref, fb_ref, wq_ref, wk_ref, wv_ref, q_ref, k_ref, v_ref):
    c = pl.program_id(1)

    @pl.when(c == 0)
    def _():
        q_ref[...] = fa_ref[...] @ wq_ref[...]
        k_ref[...] = fb_ref[...] @ wk_ref[...]

    v_ref[...] = fb_ref[...] @ wv_ref[...]


def _qkv(Fa, Fb, Wq, Wk, Wv):
    n_blk = _NA // _ROW_BLK
    row_spec = pl.BlockSpec((_ROW_BLK, _D), lambda i, c: (i, 0))
    w_spec = pl.BlockSpec((_D, _D), lambda i, c: (0, 0))
    wv_spec = pl.BlockSpec((_D, 128), lambda i, c: (0, c))
    v_spec = pl.BlockSpec((_ROW_BLK, 128), lambda i, c: (c * n_blk + i, 0))
    return pl.pallas_call(
        _qkv_body,
        grid=(n_blk, 2),
        in_specs=[row_spec, row_spec, w_spec, w_spec, wv_spec],
        out_specs=[row_spec, row_spec, v_spec],
        out_shape=[
            jax.ShapeDtypeStruct((_NA, _D), jnp.float32),
            jax.ShapeDtypeStruct((_NB, _D), jnp.float32),
            jax.ShapeDtypeStruct((2 * _NB, 128), jnp.float32),
        ],
    )(Fa, Fb, Wq, Wk, Wv)


def _k1_body(q_hbm, k_hbm, a_hbm, b_hbm, ex_hbm,
             av0, av1, bv0, bv1, qb0, qb1, kb0, kb1, exb0, exb1,
             isem0, isem1, gsem0, gsem1, ssem0, ssem1):
    isem = (isem0, isem1)
    gsem = (gsem0, gsem1)
    ssem = (ssem0, ssem1)
    cid = lax.axis_index("c")
    sid = lax.axis_index("s")
    wid = cid * _NS + sid
    nt = jnp.where(wid < _K1_EXTRA, _K1_BASE + 1, _K1_BASE)
    av = (av0, av1)
    bv = (bv0, bv1)
    qb = (qb0, qb1)
    kb = (kb0, kb1)
    exb = (exb0, exb1)

    lane = lax.iota(jnp.int32, 16)
    masks = [lane == i for i in range(16)]
    last_lane = jnp.full((16,), 15, jnp.int32)

    def coff(t):
        return (wid + _NW * t) * _C1

    def issue_idx(t, s):
        off = coff(t)
        pltpu.async_copy(a_hbm.at[pl.ds(off, _C1)], av[s], isem[s])
        pltpu.async_copy(b_hbm.at[pl.ds(off, _C1)], bv[s], isem[s])

    def wait_idx(t, s):
        off = coff(t)
        pltpu.make_async_copy(a_hbm.at[pl.ds(off, _C1)], av[s], isem[s]).wait()
        pltpu.make_async_copy(b_hbm.at[pl.ds(off, _C1)], bv[s], isem[s]).wait()

    def issue_gather(s):
        pltpu.async_copy(q_hbm.at[av[s]], qb[s], gsem[s])
        pltpu.async_copy(k_hbm.at[bv[s]], kb[s], gsem[s])

    def wait_gather(s):
        pltpu.make_async_copy(q_hbm.at[av[s]], qb[s], gsem[s]).wait()
        pltpu.make_async_copy(k_hbm.at[bv[s]], kb[s], gsem[s]).wait()

    def issue_store(t, s):
        off = coff(t)
        pltpu.async_copy(exb[s], ex_hbm.at[pl.ds(off * _H, _C1 * _H)], ssem[s])

    def wait_store(t, s):
        off = coff(t)
        pltpu.make_async_copy(
            exb[s], ex_hbm.at[pl.ds(off * _H, _C1 * _H)], ssem[s]).wait()

    issue_idx(0, 0)
    wait_idx(0, 0)
    issue_gather(0)
    issue_idx(1, 1)

    def iter_t(t, b):
        ob = 1 - b
        wait_gather(b)

        @pl.when(t >= 2)
        def _():
            wait_store(t - 2, b)

        @pl.when(t + 1 < nt)
        def _():
            wait_idx(t + 1, ob)
            issue_gather(ob)

        @pl.when(t + 2 < nt)
        def _():
            issue_idx(t + 2, b)

        qbb = qb[b]
        kbb = kb[b]
        exbb = exb[b]

        @plsc.parallel_loop(0, _C1 // 2, unroll=2)
        def pair_body(p):
            acc = jnp.zeros((16,), jnp.float32)
            for half in range(2):
                e = 2 * p + half
                for h in range(_H):
                    q0 = qbb[e, pl.ds(32 * h, 16)]
                    q1 = qbb[e, pl.ds(32 * h + 16, 16)]
                    k0 = kbb[e, pl.ds(32 * h, 16)]
                    k1 = kbb[e, pl.ds(32 * h + 16, 16)]
                    cs = jnp.cumsum(q0 * k0 + q1 * k1)
                    acc = jnp.where(masks[half * _H + h],
                                    jnp.take(cs, last_lane), acc)
            exbb[pl.ds(p * 16, 16)] = acc

        issue_store(t, b)

    def outer(u, carry):
        for b in range(2):
            t = 2 * u + b

            @pl.when(t < nt)
            def _():
                iter_t(t, b)
        return carry

    lax.fori_loop(0, (_K1_BASE + 2) // 2, outer, 0)
    wait_store(0, 0)
    wait_store(1, 1)


def _sc_logits(Q, K, a_idx, b_idx):
    f = pl.kernel(
        _k1_body,
        out_type=jax.ShapeDtypeStruct((_E * _H,), jnp.float32),
        mesh=_sc_mesh,
        compiler_params=_sc_params_tiled,
        scratch_types=(
            [pltpu.VMEM((_C1,), jnp.int32)] * 4
            + [pltpu.VMEM((_C1, _D), jnp.float32)] * 4
            + [pltpu.VMEM((_C1 * _H,), jnp.float32)] * 2
            + [pltpu.SemaphoreType.DMA] * 6
        ),
    )
    return f(Q, K, a_idx, b_idx)


def _k2_body(ex_hbm, a_hbm, b_hbm, vh_hbm, out_hbm,
             av0, av1, bv0, bv1, sav0, sav1, vb0, vb1, exb0, exb1,
             wvb0, wvb1, isem0, isem1, gsem0, gsem1, ssem0, ssem1, out_acc):
    isem = (isem0, isem1)
    gsem = (gsem0, gsem1)
    ssem = (ssem0, ssem1)
    cid = lax.axis_index("c")
    sid = lax.axis_index("s")
    nt = jnp.where(sid < _K2_EXTRA, _K2_BASE + 1, _K2_BASE)
    av = (av0, av1)
    bv = (bv0, bv1)
    sav = (sav0, sav1)
    vb = (vb0, vb1)
    exb = (exb0, exb1)
    wvb = (wvb0, wvb1)

    def zrow(r, carry):
        for j in range(8):
            wvb0[r, pl.ds(16 * j, 16)] = jnp.zeros((16,), jnp.float32)
        wvb0[r, pl.ds(_GW - 16, 16)] = jnp.zeros((16,), jnp.float32)
        return carry

    lax.fori_loop(0, _C2, zrow, 0)
    r0 = sid * _RPT
    off2 = 0
    for sz in (80,) * 7 + (65,):
        pltpu.sync_copy(wvb0.at[pl.ds(0, sz)], out_acc.at[pl.ds(r0 + off2, sz)])
        off2 += sz
    plsc.subcore_barrier()

    def coff(t):
        return (sid + _NS * t) * _C2

    def issue_idx(t, s):
        off = coff(t)
        pltpu.async_copy(a_hbm.at[pl.ds(off, _C2)], av[s], isem[s])
        pltpu.async_copy(b_hbm.at[pl.ds(off, _C2)], bv[s], isem[s])

    def wait_idx(t, s):
        off = coff(t)
        pltpu.make_async_copy(a_hbm.at[pl.ds(off, _C2)], av[s], isem[s]).wait()
        pltpu.make_async_copy(b_hbm.at[pl.ds(off, _C2)], bv[s], isem[s]).wait()

    def issue_gather(t, s):
        off = coff(t)

        def addb(i, c2):
            bv[s][pl.ds(i * 16, 16)] = bv[s][pl.ds(i * 16, 16)] + cid * _NB
            return c2

        lax.fori_loop(0, _C2 // 16, addb, 0)
        pltpu.async_copy(vh_hbm.at[bv[s]], vb[s], gsem[s])
        pltpu.async_copy(ex_hbm.at[pl.ds(off * _H, _C2 * _H)],
                         exb[s].at[pl.ds(0, _C2 * _H)], gsem[s])

    def wait_gather(t, s):
        off = coff(t)
        pltpu.make_async_copy(vh_hbm.at[bv[s]], vb[s], gsem[s]).wait()
        pltpu.make_async_copy(ex_hbm.at[pl.ds(off * _H, _C2 * _H)],
                              exb[s].at[pl.ds(0, _C2 * _H)], gsem[s]).wait()

    def issue_scatter(s):
        pltpu.async_copy(wvb[s], out_acc.at[sav[s]], ssem[s], add=True)

    def wait_scatter(s):
        pltpu.make_async_copy(wvb[s], out_acc.at[sav[s]], ssem[s]).wait()

    issue_idx(0, 0)
    wait_idx(0, 0)
    issue_gather(0, 0)
    issue_idx(1, 1)

    def iter_t(t, b):
        ob = 1 - b
        wait_gather(t, b)

        @pl.when(t >= 2)
        def _():
            wait_scatter(b)

        def cpav(i, c2):
            sav[b][pl.ds(i * 16, 16)] = av[b][pl.ds(i * 16, 16)]
            return c2

        lax.fori_loop(0, _C2 // 16, cpav, 0)

        @pl.when(t + 1 < nt)
        def _():
            wait_idx(t + 1, ob)
            issue_gather(t + 1, ob)

        @pl.when(t + 2 < nt)
        def _():
            issue_idx(t + 2, b)

        exbb = exb[b]
        vbb = vb[b]
        wvbb = wvb[b]

        @plsc.parallel_loop(0, _C2 * _H // 16, unroll=4)
        def exp_body(t2):
            v = exbb[pl.ds(t2 * 16, 16)]
            exbb[pl.ds(t2 * 16, 16)] = jnp.exp(v * _INV_SQRT_DH)

        bidx = [jnp.full((16,), i, jnp.int32) for i in range(4)]

        @plsc.parallel_loop(0, _C2, unroll=2)
        def edge_body(e):
            wvbb[e, pl.ds(0, 16)] = exbb[pl.ds(e * _H, 16)]
            exv = exbb[pl.ds(e * _H + 4 * cid, 16)]
            mv = [jnp.take(exv, bidx[i]) for i in range(4)]
            for j in range(8):
                wvbb[e, pl.ds(8 + 16 * j, 16)] = (
                    vbb[e, pl.ds(16 * j, 16)] * mv[j // 2])

        issue_scatter(b)

    def outer(u, carry):
        for b in range(2):
            t = 2 * u + b

            @pl.when(t < nt)
            def _():
                iter_t(t, b)
        return carry

    lax.fori_loop(0, (_K2_BASE + 2) // 2, outer, 0)
    wait_scatter(0)
    wait_scatter(1)
    plsc.subcore_barrier()

    off2 = 0
    for sz in (80,) * 7 + (65,):
        pltpu.sync_copy(out_acc.at[pl.ds(r0 + off2, sz)], wvb0.at[pl.ds(0, sz)])
        pltpu.sync_copy(wvb0.at[pl.ds(0, sz)],
                        out_hbm.at[cid, pl.ds(r0 + off2, sz)])
        off2 += sz


def _sc_scatter(ex, a_idx, b_idx, Vh):
    f = pl.kernel(
        _k2_body,
        out_type=jax.ShapeDtypeStruct((2, _NA, _GW), jnp.float32),
        mesh=_sc_mesh,
        compiler_params=_sc_params,
        scratch_types=(
            [pltpu.VMEM((_C2,), jnp.int32)] * 6
            + [pltpu.VMEM((_C2, 128), jnp.float32)] * 2
            + [pltpu.VMEM((_C2 * _H + 16,), jnp.float32)] * 2
            + [pltpu.VMEM((_C2, _GW), jnp.float32)] * 2
            + [pltpu.SemaphoreType.DMA] * 6
            + [pltpu.VMEM_SHARED((_NA, _GW), jnp.float32)]
        ),
    )
    return f(ex, a_idx, b_idx, Vh)


def _final_body(fa_ref, g0_ref, g1_ref, wp_ref, y_ref):
    g0 = g0_ref[0]
    g1 = g1_ref[0]
    s = g0[:, 0:8]
    s = jnp.where(s > 0.0, s, 1.0)
    o = jnp.concatenate([g0[:, 8:_GW], g1[:, 8:_GW]], axis=1)
    o = (o.reshape(_ROW_BLK, _H, _DH) / s[:, :, None]).reshape(_ROW_BLK, _D)
    y_ref[...] = fa_ref[...] + o @ wp_ref[...]


def _final(Fa, g, Wp):
    n_blk = _NA // _ROW_BLK
    row_spec = pl.BlockSpec((_ROW_BLK, _D), lambda i: (i, 0))
    g0_spec = pl.BlockSpec((1, _ROW_BLK, _GW), lambda i: (0, i, 0))
    g1_spec = pl.BlockSpec((1, _ROW_BLK, _GW), lambda i: (1, i, 0))
    w_spec = pl.BlockSpec((_D, _D), lambda i: (0, 0))
    return pl.pallas_call(
        _final_body,
        grid=(n_blk,),
        in_specs=[row_spec, g0_spec, g1_spec, w_spec],
        out_specs=row_spec,
        out_shape=jax.ShapeDtypeStruct((_NA, _D), jnp.float32),
    )(Fa, g, g, Wp)


def kernel(Fa, Fb, a_idx, b_idx, Wq, Wk, Wv, Wp):
    a_idx = a_idx.astype(jnp.int32)
    b_idx = b_idx.astype(jnp.int32)
    Q, K, Vh = _qkv(Fa, Fb, Wq, Wk, Wv)
    ex = _sc_logits(Q, K, a_idx, b_idx)
    g = _sc_scatter(ex, a_idx, b_idx, Vh)
    return _final(Fa, g, Wp)
